# baseline (speedup 1.0000x reference)
import functools

import jax
import jax.numpy as jnp
from jax import lax
from jax.experimental import pallas as pl
from jax.experimental.pallas import tpu as pltpu
from jax.experimental.pallas import tpu_sc as plsc

VOCAB = 1000000
EMBED = 64
BATCH = 16384
NEG = 10
SLOTS = 16
ROW = 128
GPR = ROW // SLOTS

NUM_CORES = 2
NUM_SUBCORES = 16
LANES = 16
NW = NUM_CORES * NUM_SUBCORES
PER_W = BATCH // NW
CHUNK = 64
NCHUNKS = PER_W // CHUNK


_mesh = plsc.VectorSubcoreMesh(core_axis_name="c", subcore_axis_name="s")


@functools.partial(
    pl.kernel,
    out_type=jax.ShapeDtypeStruct((BATCH // GPR, ROW), jnp.float32),
    mesh=_mesh,
    compiler_params=pltpu.CompilerParams(needs_layout_passes=False),
    scratch_types=[
        pltpu.VMEM((PER_W,), jnp.int32),
        pltpu.VMEM((PER_W,), jnp.int32),
        pltpu.VMEM((PER_W,), jnp.int32),
        pltpu.VMEM((PER_W,), jnp.int32),
        pltpu.VMEM((NEG, PER_W), jnp.int32),
        pltpu.VMEM((NEG, PER_W), jnp.int32),
        pltpu.VMEM((PER_W,), jnp.int32),
        pltpu.VMEM((PER_W,), jnp.int32),
        pltpu.VMEM((CHUNK, ROW), jnp.float32),
        pltpu.VMEM((CHUNK, ROW), jnp.float32),
        pltpu.VMEM((NEG, CHUNK, ROW), jnp.float32),
        pltpu.VMEM((CHUNK // GPR, ROW), jnp.float32),
        pltpu.SemaphoreType.DMA,
    ],
)
def _scores_sc(iw_hbm, cw_hbm, negt_hbm, win_hbm, wctx_hbm, out_hbm,
               iwh, iwo, cwh, cwo, ngh, ngo, rawa, rawb,
               ei, ec, en, sc_v, sem):
    wid = lax.axis_index("s") * NUM_CORES + lax.axis_index("c")
    wbase = wid * PER_W
    lane = lax.iota(jnp.int32, LANES)

    pltpu.sync_copy(iw_hbm.at[pl.ds(wbase, PER_W)], rawa)
    pltpu.sync_copy(cw_hbm.at[pl.ds(wbase, PER_W)], rawb)
    for t in range(PER_W // LANES):
        s = pl.ds(t * LANES, LANES)
        v = rawa[s]
        iwh[s] = v >> 1
        iwo[s] = (v & 1) * EMBED
        v = rawb[s]
        cwh[s] = v >> 1
        cwo[s] = (v & 1) * EMBED
    for k in range(NEG):
        pltpu.sync_copy(negt_hbm.at[pl.ds(k * BATCH + wbase, PER_W)], rawa)
        for t in range(PER_W // LANES):
            s = pl.ds(t * LANES, LANES)
            v = rawa[s]
            ngh[k, s] = v >> 1
            ngo[k, s] = (v & 1) * EMBED

    def chunk_body(ci, _):
        cbase = ci * CHUNK
        copies = [
            pltpu.async_copy(win_hbm.at[iwh.at[pl.ds(cbase, CHUNK)]], ei, sem),
            pltpu.async_copy(wctx_hbm.at[cwh.at[pl.ds(cbase, CHUNK)]], ec, sem),
        ]
        for k in range(NEG):
            copies.append(pltpu.async_copy(
                wctx_hbm.at[ngh.at[k, pl.ds(cbase, CHUNK)]], en.at[k], sem))
        for c in copies:
            c.wait()

        for g in range(CHUNK // LANES):
            rows = lane + g * LANES
            sg = pl.ds(cbase + g * LANES, LANES)
            coli = iwo[sg]
            colc = cwo[sg]
            coln = [ngo[k, sg] for k in range(NEG)]
            kk = [jnp.full((LANES,), k, jnp.int32) for k in range(NEG)]

            def d_body(dd, accs):
                ei_d = plsc.load_gather(ei, [rows, coli + dd])
                ec_d = plsc.load_gather(ec, [rows, colc + dd])
                pos = accs[0] + ei_d * ec_d
                new = [pos]
                for k in range(NEG):
                    en_d = plsc.load_gather(en, [kk[k], rows, coln[k] + dd])
                    new.append(accs[k + 1] - en_d * ei_d)
                return tuple(new)

            zero = jnp.zeros((LANES,), jnp.float32)
            accs = lax.fori_loop(0, EMBED, d_body, (zero,) * (NEG + 1))
            srow = rows >> 3
            scol0 = (rows & 7) << 4
            for s in range(NEG + 1):
                plsc.store_scatter(sc_v, [srow, scol0 + s], accs[s])
        orow = pl.multiple_of((wbase + cbase) // GPR, 8)
        pltpu.sync_copy(sc_v, out_hbm.at[pl.ds(orow, CHUNK // GPR)])
        return 0

    lax.fori_loop(0, NCHUNKS, chunk_body, 0)


def _loss_tc(scores_ref, out_ref):
    x = scores_ref[...]
    col = lax.broadcasted_iota(jnp.int32, x.shape, 1)
    ls = jnp.minimum(x, 0.0) - jnp.log1p(jnp.exp(-jnp.abs(x)))
    m = jnp.where(col % SLOTS < NEG + 1, ls, 0.0)
    out_ref[0, 0] = -jnp.sum(m) / BATCH


def kernel(input_word, context_word, W_in, W_ctx):
    neg = jax.random.randint(jax.random.key(1234), (BATCH, NEG), 0, VOCAB)
    negt = neg.astype(jnp.int32).T.reshape(-1)
    iw = input_word.astype(jnp.int32)
    cw = context_word.astype(jnp.int32)
    win2 = W_in.reshape(VOCAB // 2, ROW)
    wctx2 = W_ctx.reshape(VOCAB // 2, ROW)
    scores = _scores_sc(iw, cw, negt, win2, wctx2)
    loss = pl.pallas_call(
        _loss_tc,
        out_shape=jax.ShapeDtypeStruct((1, 1), jnp.float32),
        out_specs=pl.BlockSpec(memory_space=pltpu.SMEM),
    )(scores)
    return loss[0, 0]


# speedup vs baseline: 1.0114x; 1.0114x over previous
import functools

import jax
import jax.numpy as jnp
from jax import lax
from jax.experimental import pallas as pl
from jax.experimental.pallas import tpu as pltpu
from jax.experimental.pallas import tpu_sc as plsc

VOCAB = 1000000
EMBED = 64
BATCH = 16384
NEG = 10
SLOTS = 16
ROW = 128
GPR = ROW // SLOTS

NUM_CORES = 2
NUM_SUBCORES = 16
LANES = 16
NW = NUM_CORES * NUM_SUBCORES
PER_W = BATCH // NW
CHUNK = 64
NCHUNKS = PER_W // CHUNK


_mesh = plsc.VectorSubcoreMesh(core_axis_name="c", subcore_axis_name="s")


@functools.partial(
    pl.kernel,
    out_type=jax.ShapeDtypeStruct((BATCH // GPR, ROW), jnp.float32),
    mesh=_mesh,
    compiler_params=pltpu.CompilerParams(needs_layout_passes=False),
    scratch_types=[
        pltpu.VMEM((PER_W,), jnp.int32),
        pltpu.VMEM((PER_W,), jnp.int32),
        pltpu.VMEM((PER_W,), jnp.int32),
        pltpu.VMEM((PER_W,), jnp.int32),
        pltpu.VMEM((NEG, PER_W), jnp.int32),
        pltpu.VMEM((NEG, PER_W), jnp.int32),
        pltpu.VMEM((PER_W,), jnp.int32),
        pltpu.VMEM((PER_W,), jnp.int32),
        pltpu.VMEM((CHUNK, ROW), jnp.float32),
        pltpu.VMEM((CHUNK, ROW), jnp.float32),
        pltpu.VMEM((NEG, CHUNK, ROW), jnp.float32),
        pltpu.VMEM((CHUNK // GPR, ROW), jnp.float32),
        pltpu.SemaphoreType.DMA,
    ],
)
def _scores_sc(iw_hbm, cw_hbm, negt_hbm, win_hbm, wctx_hbm, out_hbm,
               iwh, iwo, cwh, cwo, ngh, ngo, rawa, rawb,
               ei, ec, en, sc_v, sem):
    wid = lax.axis_index("s") * NUM_CORES + lax.axis_index("c")
    wbase = wid * PER_W
    lane = lax.iota(jnp.int32, LANES)

    pltpu.sync_copy(iw_hbm.at[pl.ds(wbase, PER_W)], rawa)
    pltpu.sync_copy(cw_hbm.at[pl.ds(wbase, PER_W)], rawb)
    for t in range(PER_W // LANES):
        s = pl.ds(t * LANES, LANES)
        v = rawa[s]
        iwh[s] = v >> 1
        iwo[s] = (v & 1) * EMBED
        v = rawb[s]
        cwh[s] = v >> 1
        cwo[s] = (v & 1) * EMBED
    for k in range(NEG):
        pltpu.sync_copy(negt_hbm.at[pl.ds(k * BATCH + wbase, PER_W)], rawa)
        for t in range(PER_W // LANES):
            s = pl.ds(t * LANES, LANES)
            v = rawa[s]
            ngh[k, s] = v >> 1
            ngo[k, s] = (v & 1) * EMBED

    def chunk_body(ci, _):
        cbase = ci * CHUNK
        copies = [
            pltpu.async_copy(win_hbm.at[iwh.at[pl.ds(cbase, CHUNK)]], ei, sem),
            pltpu.async_copy(wctx_hbm.at[cwh.at[pl.ds(cbase, CHUNK)]], ec, sem),
        ]
        for k in range(NEG):
            copies.append(pltpu.async_copy(
                wctx_hbm.at[ngh.at[k, pl.ds(cbase, CHUNK)]], en.at[k], sem))
        for c in copies:
            c.wait()

        nq = EMBED // LANES

        def g_body(g, _):
            gb = g * LANES
            vio = iwo[pl.ds(cbase + gb, LANES)]
            vco = cwo[pl.ds(cbase + gb, LANES)]
            vno = [ngo[k, pl.ds(cbase + gb, LANES)] for k in range(NEG)]
            for l in range(LANES):
                j = gb + l
                offi = vio[l]
                offc = vco[l]
                eir = [ei[j, pl.ds(offi + q * LANES, LANES)]
                       for q in range(nq)]
                ecr = [ec[j, pl.ds(offc + q * LANES, LANES)]
                       for q in range(nq)]
                p = eir[0] * ecr[0]
                for q in range(1, nq):
                    p = p + eir[q] * ecr[q]
                vals = jnp.where(lane == 0, jnp.sum(p), 0.0)
                for k in range(NEG):
                    offn = vno[k][l]
                    enr = [en[k, j, pl.ds(offn + q * LANES, LANES)]
                           for q in range(nq)]
                    p = eir[0] * enr[0]
                    for q in range(1, nq):
                        p = p + eir[q] * enr[q]
                    vals = jnp.where(lane == k + 1, -jnp.sum(p), vals)
                sc_v[j // GPR, pl.ds((j % GPR) * SLOTS, SLOTS)] = vals
            return 0

        lax.fori_loop(0, CHUNK // LANES, g_body, 0)
        orow = pl.multiple_of((wbase + cbase) // GPR, 8)
        pltpu.sync_copy(sc_v, out_hbm.at[pl.ds(orow, CHUNK // GPR)])
        return 0

    lax.fori_loop(0, NCHUNKS, chunk_body, 0)


def _loss_tc(scores_ref, out_ref):
    x = scores_ref[...]
    col = lax.broadcasted_iota(jnp.int32, x.shape, 1)
    ls = jnp.minimum(x, 0.0) - jnp.log1p(jnp.exp(-jnp.abs(x)))
    m = jnp.where(col % SLOTS < NEG + 1, ls, 0.0)
    out_ref[0, 0] = -jnp.sum(m) / BATCH


def kernel(input_word, context_word, W_in, W_ctx):
    neg = jax.random.randint(jax.random.key(1234), (BATCH, NEG), 0, VOCAB)
    negt = neg.astype(jnp.int32).T.reshape(-1)
    iw = input_word.astype(jnp.int32)
    cw = context_word.astype(jnp.int32)
    one = (iw[0] * 0 + 1).astype(jnp.float32)
    win2 = W_in.reshape(VOCAB // 2, ROW) * one
    wctx2 = W_ctx.reshape(VOCAB // 2, ROW) * one
    scores = _scores_sc(iw, cw, negt, win2, wctx2)
    loss = pl.pallas_call(
        _loss_tc,
        out_shape=jax.ShapeDtypeStruct((1, 1), jnp.float32),
        out_specs=pl.BlockSpec(memory_space=pltpu.SMEM),
    )(scores)
    return loss[0, 0]
